# Initial kernel scaffold; baseline (speedup 1.0000x reference)
import jax
import jax.numpy as jnp
from jax.experimental import pallas as pl


def kernel(x, edge_index, W_self0, W_neigh0, W_self1, W_neigh1, W_self2, W_neigh2):
    raise NotImplementedError("write your pallas kernel here")


# speedup vs baseline: 3.8312x; 3.8312x over previous
import functools

import jax
import jax.numpy as jnp
from jax import lax
from jax.experimental import pallas as pl
from jax.experimental.pallas import tpu as pltpu
from jax.experimental.pallas import tpu_sc as plsc

N = 10000
E = 320000
D = 128

NC = 2
NS = 16
NW = NC * NS
K = 128
CHUNKS = 79
EPT = CHUNKS * K
E_PAD = NW * EPT
NPAD = 10112
ZROWS = NPAD // NS

_MESH = plsc.VectorSubcoreMesh(core_axis_name="c", subcore_axis_name="s")


@functools.partial(
    pl.kernel, mesh=_MESH,
    out_type=jax.ShapeDtypeStruct((NW, ZROWS, D), jnp.float32),
    scratch_types=[
        pltpu.VMEM((K,), jnp.int32),
        pltpu.VMEM((K,), jnp.int32),
        pltpu.VMEM((K, D), jnp.float32),
        pltpu.VMEM_SHARED((NPAD, D), jnp.float32),
        pltpu.SemaphoreType.DMA,
    ],
)
def _seg(table_hbm, src_hbm, dst_hbm, zeros_hbm, out_hbm,
         src_v, dst_v, rows_v, acc, sem):
    c = lax.axis_index("c")
    s = lax.axis_index("s")
    wid = c * NS + s
    pltpu.sync_copy(zeros_hbm.at[pl.ds(s * ZROWS, ZROWS)],
                    acc.at[pl.ds(s * ZROWS, ZROWS)])
    plsc.subcore_barrier()

    def body(i, carry):
        base = wid * EPT + i * K
        pltpu.sync_copy(src_hbm.at[pl.ds(base, K)], src_v)
        pltpu.sync_copy(dst_hbm.at[pl.ds(base, K)], dst_v)
        pltpu.async_copy(table_hbm.at[src_v], rows_v, sem).wait()
        pltpu.sync_copy(rows_v, acc.at[dst_v], add=True)
        return carry

    lax.fori_loop(0, CHUNKS, body, 0)
    plsc.subcore_barrier()
    pltpu.sync_copy(acc.at[pl.ds(s * ZROWS, ZROWS)], out_hbm.at[wid])


@functools.partial(
    pl.kernel, mesh=_MESH,
    out_type=jax.ShapeDtypeStruct((NW, ZROWS, D), jnp.float32),
    scratch_types=[
        pltpu.VMEM((K,), jnp.int32),
        pltpu.VMEM((K, D), jnp.float32),
        pltpu.VMEM_SHARED((NPAD, D), jnp.float32),
    ],
)
def _deg(ones_hbm, dst_hbm, zeros_hbm, out_hbm, dst_v, ones_v, acc):
    c = lax.axis_index("c")
    s = lax.axis_index("s")
    wid = c * NS + s
    pltpu.sync_copy(zeros_hbm.at[pl.ds(s * ZROWS, ZROWS)],
                    acc.at[pl.ds(s * ZROWS, ZROWS)])
    pltpu.sync_copy(ones_hbm, ones_v)
    plsc.subcore_barrier()

    def body(i, carry):
        base = wid * EPT + i * K
        pltpu.sync_copy(dst_hbm.at[pl.ds(base, K)], dst_v)
        pltpu.sync_copy(ones_v, acc.at[dst_v], add=True)
        return carry

    lax.fori_loop(0, CHUNKS, body, 0)
    plsc.subcore_barrier()
    pltpu.sync_copy(acc.at[pl.ds(s * ZROWS, ZROWS)], out_hbm.at[wid])


def _layer0_body(x_ref, a_ref, b_ref, dega_ref, degb_ref, ws_ref, wn_ref,
                 h_ref, inv_ref):
    inv = 1.0 / jnp.maximum(dega_ref[...] + degb_ref[...], 1.0)
    mean = (a_ref[...] + b_ref[...]) * inv
    h = (jnp.dot(x_ref[...], ws_ref[...], preferred_element_type=jnp.float32)
         + jnp.dot(mean, wn_ref[...], preferred_element_type=jnp.float32))
    h_ref[...] = jnp.maximum(h, 0.0)
    inv_ref[...] = inv


def _layerN_body(h_ref, a_ref, b_ref, inv_ref, ws_ref, wn_ref, o_ref, *, relu):
    mean = (a_ref[...] + b_ref[...]) * inv_ref[...]
    o = (jnp.dot(h_ref[...], ws_ref[...], preferred_element_type=jnp.float32)
         + jnp.dot(mean, wn_ref[...], preferred_element_type=jnp.float32))
    if relu:
        o = jnp.maximum(o, 0.0)
    o_ref[...] = o


_BLK = 1000
_GRID = N // _BLK


def _row_spec(w):
    return pl.BlockSpec((_BLK, w), lambda i: (i, 0))


def _full_spec(r, w):
    return pl.BlockSpec((r, w), lambda i: (0, 0))


def _tc_layer0(x, accA, accB, degA, degB, ws, wn):
    return pl.pallas_call(
        _layer0_body,
        grid=(_GRID,),
        in_specs=[_row_spec(128), _row_spec(128), _row_spec(128),
                  _row_spec(128), _row_spec(128),
                  _full_spec(128, 128), _full_spec(128, 128)],
        out_specs=[_row_spec(128), _row_spec(128)],
        out_shape=[jax.ShapeDtypeStruct((N, 128), jnp.float32),
                   jax.ShapeDtypeStruct((N, 128), jnp.float32)],
    )(x, accA, accB, degA, degB, ws, wn)


def _tc_layerN(h, accA, accB, invb, ws, wn, relu):
    return pl.pallas_call(
        functools.partial(_layerN_body, relu=relu),
        grid=(_GRID,),
        in_specs=[_row_spec(128), _row_spec(128), _row_spec(128),
                  _row_spec(128), _full_spec(128, 128), _full_spec(128, 128)],
        out_specs=_row_spec(128),
        out_shape=jax.ShapeDtypeStruct((N, 128), jnp.float32),
    )(h, accA, accB, invb, ws, wn)


def kernel(x, edge_index, W_self0, W_neigh0, W_self1, W_neigh1, W_self2, W_neigh2):
    src = edge_index[0]
    dst = edge_index[1]
    pad = E_PAD - E
    srcp = jnp.pad(src, (0, pad))
    dstp = jnp.pad(dst, (0, pad), constant_values=N)
    zeros = jnp.zeros((NPAD, D), jnp.float32)
    ones = jnp.ones((K, D), jnp.float32)

    degp = _deg(ones, dstp, zeros).reshape(NC, NPAD, D)[:, :N]

    acc0 = _seg(x, srcp, dstp, zeros).reshape(NC, NPAD, D)[:, :N]
    h1, invb = _tc_layer0(x, acc0[0], acc0[1], degp[0], degp[1],
                          W_self0, W_neigh0)

    acc1 = _seg(h1, srcp, dstp, zeros).reshape(NC, NPAD, D)[:, :N]
    h2 = _tc_layerN(h1, acc1[0], acc1[1], invb, W_self1, W_neigh1, relu=True)

    acc2 = _seg(h2, srcp, dstp, zeros).reshape(NC, NPAD, D)[:, :N]
    ws2 = jnp.pad(W_self2, ((0, 0), (0, 128 - 40)))
    wn2 = jnp.pad(W_neigh2, ((0, 0), (0, 128 - 40)))
    h3 = _tc_layerN(h2, acc2[0], acc2[1], invb, ws2, wn2, relu=False)
    return h3[:, :40]
